# Initial kernel scaffold; baseline (speedup 1.0000x reference)
import jax
import jax.numpy as jnp
from jax.experimental import pallas as pl


def kernel(x, Wr, br, Ws, bs, Wdw, bdw, Wpw, bpw, Wfc, bfc):
    raise NotImplementedError("write your pallas kernel here")


# speedup vs baseline: 4.8400x; 4.8400x over previous
import functools

import jax
import jax.numpy as jnp
from jax import lax
from jax.experimental import pallas as pl
from jax.experimental.pallas import tpu as pltpu

E = 8
B = 32
H = 224
HO = 112
WP = 128
NUM_CLASSES = 1000
K = 3 * H * H
KC = 8
KCHUNK = K // KC
_PREC = lax.Precision.HIGHEST


def _roll(v, shift, axis):
    return pltpu.roll(v, shift % v.shape[axis], axis)


def _router_kernel(x_ref, w_ref, br_ref, r_ref, sel0_ref, gate_ref,
                   idx_ref, loss_ref):
    k = pl.program_id(0)

    @pl.when(k == 0)
    def _init():
        r_ref[...] = jnp.zeros_like(r_ref)

    part = lax.dot_general(
        x_ref[...], w_ref[...], (((1,), (1,)), ((), ())),
        precision=_PREC, preferred_element_type=jnp.float32)
    r_ref[...] += part

    @pl.when(k == KC - 1)
    def _finish():
        r = r_ref[...] + 3136.0 * br_ref[...]
        m = jnp.max(r, axis=1, keepdims=True)
        ex = jnp.exp(r - m)
        s = jnp.sum(ex, axis=1, keepdims=True)
        sel = ex / s
        gate_ref[...] = jnp.max(sel, axis=1, keepdims=True)
        idx = jnp.argmax(r, axis=1)
        idx_ref[...] = idx[:, None].astype(jnp.int32)
        eiota = lax.broadcasted_iota(jnp.int32, (B, E), 1)
        sel0 = (eiota == idx[:, None]).astype(jnp.float32)
        sel0_ref[...] = sel0
        density = jnp.mean(sel0, axis=0)
        proxy = jnp.mean(sel, axis=0)
        loss = jnp.sum(proxy * density) * float(E)
        loss_ref[...] = jnp.reshape(loss, (1, 1))


def _expert_kernel(idx_ref, gate_ref, x_ref, ws_ref, bs_ref, wdw_ref,
                   bdw_ref, wpw_ref, bpw_ref, wfc_ref, bfc_ref, out_ref):
    del idx_ref
    b = pl.program_id(0)
    g = gate_ref[b]
    X = x_ref[0]

    riota = lax.broadcasted_iota(jnp.int32, (12 * HO, WP), 0)
    vmask = (riota % HO != HO - 1).astype(jnp.float32)
    Xv = _roll(X, -1, 0) * vmask
    Xh = _roll(X, -1, 1)
    Xvh = _roll(Xv, -1, 1)
    variants = {(0, 0): X, (1, 0): Xv, (0, 1): Xh, (1, 1): Xvh}

    acc = jnp.zeros((32, HO, WP), jnp.float32)
    for ki in range(3):
        for kj in range(3):
            V = variants[(ki // 2, kj // 2)]
            pi, pj = ki % 2, kj % 2
            for c in range(3):
                p = c * 4 + pi * 2 + pj
                plane = V[p * HO:(p + 1) * HO, :]
                wv = ws_ref[0, c * 9 + ki * 3 + kj, :]
                acc = acc + wv[:, None, None] * plane[None, :, :]
    colmask = (lax.broadcasted_iota(jnp.int32, (1, 1, WP), 2) < HO
               ).astype(jnp.float32)
    bs = bs_ref[0, 0, :]
    h1 = jnp.maximum(acc + bs[:, None, None], 0.0) * colmask

    row3 = lax.broadcasted_iota(jnp.int32, (32, HO, WP), 1)
    top = (row3 != 0).astype(jnp.float32)
    bot = (row3 != HO - 1).astype(jnp.float32)
    vvar = {
        -1: _roll(h1, 1, 1) * top,
        0: h1,
        1: _roll(h1, -1, 1) * bot,
    }
    acc2 = jnp.zeros((32, HO, WP), jnp.float32)
    for di in (-1, 0, 1):
        for dj in (-1, 0, 1):
            Vt = vvar[di]
            if dj:
                Vt = _roll(Vt, -dj, 2)
            wv = wdw_ref[0, (di + 1) * 3 + (dj + 1), :]
            acc2 = acc2 + wv[:, None, None] * Vt
    bdw = bdw_ref[0, 0, :]
    h2 = jnp.maximum(acc2 + bdw[:, None, None], 0.0) * colmask

    h2f = h2.reshape(32, HO * WP)
    h3 = lax.dot_general(
        wpw_ref[0], h2f, (((1,), (0,)), ((), ())),
        precision=_PREC, preferred_element_type=jnp.float32)
    bpw = bpw_ref[0, 0, :]
    h3 = jnp.maximum(h3 + bpw[:, None], 0.0)

    fmask = (lax.broadcasted_iota(jnp.int32, (1, HO * WP), 1) % WP < HO
             ).astype(jnp.float32)
    pvec = jnp.sum(h3 * fmask, axis=1) / float(HO * HO)

    logits = lax.dot_general(
        pvec[None, :], wfc_ref[0], (((1,), (0,)), ((), ())),
        precision=_PREC, preferred_element_type=jnp.float32)
    logits = logits + bfc_ref[0, 0, :][None, :]
    z = g * logits
    zm = z - jnp.max(z, axis=1, keepdims=True)
    ez = jnp.exp(zm)
    out_ref[0] = ez / jnp.sum(ez, axis=1, keepdims=True)


@functools.partial(jax.jit, static_argnames=())
def kernel(x, Wr, br, Ws, bs, Wdw, bdw, Wpw, bpw, Wfc, bfc):
    xf = x.reshape(B, K)
    wt = jnp.tile(Wr, (1, 1, 56, 56)).reshape(E, K)

    r, sel0, gate, idx, loss = pl.pallas_call(
        _router_kernel,
        grid=(KC,),
        in_specs=[
            pl.BlockSpec((B, KCHUNK), lambda k: (0, k)),
            pl.BlockSpec((E, KCHUNK), lambda k: (0, k)),
            pl.BlockSpec((1, E), lambda k: (0, 0)),
        ],
        out_specs=[
            pl.BlockSpec((B, E), lambda k: (0, 0)),
            pl.BlockSpec((B, E), lambda k: (0, 0)),
            pl.BlockSpec((B, 1), lambda k: (0, 0)),
            pl.BlockSpec((B, 1), lambda k: (0, 0)),
            pl.BlockSpec((1, 1), lambda k: (0, 0)),
        ],
        out_shape=[
            jax.ShapeDtypeStruct((B, E), jnp.float32),
            jax.ShapeDtypeStruct((B, E), jnp.float32),
            jax.ShapeDtypeStruct((B, 1), jnp.float32),
            jax.ShapeDtypeStruct((B, 1), jnp.int32),
            jax.ShapeDtypeStruct((1, 1), jnp.float32),
        ],
    )(xf, wt, br.reshape(1, E))
    del r

    x4 = x.reshape(B, 3, HO, 2, HO, 2).transpose(0, 1, 3, 5, 2, 4)
    x4 = x4.reshape(B, 12, HO, HO)
    x4p = jnp.pad(x4, ((0, 0), (0, 0), (0, 0), (0, WP - HO))
                  ).reshape(B, 12 * HO, WP)

    wsr = Ws.transpose(0, 2, 3, 4, 1).reshape(E, 27, 32)
    wdwr = Wdw.reshape(E, 32, 9).transpose(0, 2, 1)
    wpwr = Wpw.reshape(E, 64, 32)
    bs3 = bs.reshape(E, 1, 32)
    bdw3 = bdw.reshape(E, 1, 32)
    bpw3 = bpw.reshape(E, 1, 64)
    bfc3 = bfc.reshape(E, 1, NUM_CLASSES)

    idx_flat = idx.reshape(B)
    gate_flat = gate.reshape(B)

    grid_spec = pltpu.PrefetchScalarGridSpec(
        num_scalar_prefetch=2,
        grid=(B,),
        in_specs=[
            pl.BlockSpec((1, 12 * HO, WP), lambda b, i, g: (b, 0, 0)),
            pl.BlockSpec((1, 27, 32), lambda b, i, g: (i[b], 0, 0)),
            pl.BlockSpec((1, 1, 32), lambda b, i, g: (i[b], 0, 0)),
            pl.BlockSpec((1, 9, 32), lambda b, i, g: (i[b], 0, 0)),
            pl.BlockSpec((1, 1, 32), lambda b, i, g: (i[b], 0, 0)),
            pl.BlockSpec((1, 64, 32), lambda b, i, g: (i[b], 0, 0)),
            pl.BlockSpec((1, 1, 64), lambda b, i, g: (i[b], 0, 0)),
            pl.BlockSpec((1, 64, NUM_CLASSES), lambda b, i, g: (i[b], 0, 0)),
            pl.BlockSpec((1, 1, NUM_CLASSES), lambda b, i, g: (i[b], 0, 0)),
        ],
        out_specs=pl.BlockSpec((1, 1, NUM_CLASSES), lambda b, i, g: (b, 0, 0)),
    )
    out3 = pl.pallas_call(
        _expert_kernel,
        grid_spec=grid_spec,
        out_shape=jax.ShapeDtypeStruct((B, 1, NUM_CLASSES), jnp.float32),
    )(idx_flat, gate_flat, x4p, wsr, bs3, wdwr, bdw3, wpwr, bpw3, Wfc, bfc3)

    output = out3.reshape(B, NUM_CLASSES)
    return (output, sel0, loss.reshape(()), 0)


# speedup vs baseline: 5.5680x; 1.1504x over previous
import functools

import jax
import jax.numpy as jnp
from jax import lax
from jax.experimental import pallas as pl
from jax.experimental.pallas import tpu as pltpu

E = 8
B = 32
H = 224
HO = 112
WP = 128
NUM_CLASSES = 1000
K = 3 * H * H
KC = 8
KCHUNK = K // KC
_PREC = lax.Precision.HIGHEST


def _roll(v, shift, axis):
    return pltpu.roll(v, shift % v.shape[axis], axis)


def _router_kernel(x_ref, w_ref, br_ref, r_ref, sel0_ref, gate_ref,
                   idx_ref, loss_ref):
    k = pl.program_id(0)

    @pl.when(k == 0)
    def _init():
        r_ref[...] = jnp.zeros_like(r_ref)

    part = lax.dot_general(
        x_ref[...], w_ref[...], (((1,), (1,)), ((), ())),
        precision=_PREC, preferred_element_type=jnp.float32)
    r_ref[...] += part

    @pl.when(k == KC - 1)
    def _finish():
        r = r_ref[...] + 3136.0 * br_ref[...]
        m = jnp.max(r, axis=1, keepdims=True)
        ex = jnp.exp(r - m)
        s = jnp.sum(ex, axis=1, keepdims=True)
        sel = ex / s
        gate_ref[...] = jnp.max(sel, axis=1, keepdims=True)
        idx = jnp.argmax(r, axis=1)
        idx_ref[...] = idx[:, None].astype(jnp.int32)
        eiota = lax.broadcasted_iota(jnp.int32, (B, E), 1)
        sel0 = (eiota == idx[:, None]).astype(jnp.float32)
        sel0_ref[...] = sel0
        density = jnp.mean(sel0, axis=0)
        proxy = jnp.mean(sel, axis=0)
        loss = jnp.sum(proxy * density) * float(E)
        loss_ref[...] = jnp.reshape(loss, (1, 1))


def _expert_kernel(idx_ref, gate_ref, x_ref, ws_ref, bs_ref, wdw_ref,
                   bdw_ref, wpw_ref, bpw_ref, wfc_ref, bfc_ref, out_ref):
    del idx_ref
    b = pl.program_id(0)
    g = gate_ref[b]
    X = x_ref[0]

    riota = lax.broadcasted_iota(jnp.int32, (12 * HO, WP), 0)
    vmask = (riota % HO != HO - 1).astype(jnp.float32)
    Xv = _roll(X, -1, 0) * vmask
    Xh = _roll(X, -1, 1)
    Xvh = _roll(Xv, -1, 1)
    variants = {(0, 0): X, (1, 0): Xv, (0, 1): Xh, (1, 1): Xvh}

    taps = []
    for ki in range(3):
        for kj in range(3):
            V = variants[(ki // 2, kj // 2)]
            pi, pj = ki % 2, kj % 2
            for c in range(3):
                p = c * 4 + pi * 2 + pj
                taps.append(V[p * HO:(p + 1) * HO, :])
    P = jnp.stack(taps, axis=0).reshape(27, HO * WP)
    h1f = lax.dot_general(ws_ref[0], P, (((1,), (0,)), ((), ())),
                          precision=_PREC, preferred_element_type=jnp.float32)
    fmask = (lax.broadcasted_iota(jnp.int32, (1, HO * WP), 1) % WP < HO
             ).astype(jnp.float32)
    colmask = (lax.broadcasted_iota(jnp.int32, (1, 1, WP), 2) < HO
               ).astype(jnp.float32)
    bs = bs_ref[0, 0, :]
    h1 = (jnp.maximum(h1f + bs[:, None], 0.0) * fmask).reshape(32, HO, WP)

    row3 = lax.broadcasted_iota(jnp.int32, (32, HO, WP), 1)
    top = (row3 != 0).astype(jnp.float32)
    bot = (row3 != HO - 1).astype(jnp.float32)
    vvar = {
        -1: _roll(h1, 1, 1) * top,
        0: h1,
        1: _roll(h1, -1, 1) * bot,
    }
    acc2 = jnp.zeros((32, HO, WP), jnp.float32)
    for di in (-1, 0, 1):
        for dj in (-1, 0, 1):
            Vt = vvar[di]
            if dj:
                Vt = _roll(Vt, -dj, 2)
            wv = wdw_ref[0, (di + 1) * 3 + (dj + 1), :]
            acc2 = acc2 + wv[:, None, None] * Vt
    bdw = bdw_ref[0, 0, :]
    h2 = jnp.maximum(acc2 + bdw[:, None, None], 0.0) * colmask

    h2f = h2.reshape(32, HO * WP)
    h3 = lax.dot_general(
        wpw_ref[0], h2f, (((1,), (0,)), ((), ())),
        precision=_PREC, preferred_element_type=jnp.float32)
    bpw = bpw_ref[0, 0, :]
    h3 = jnp.maximum(h3 + bpw[:, None], 0.0)

    pvec = jnp.sum(h3 * fmask, axis=1) / float(HO * HO)

    logits = lax.dot_general(
        pvec[None, :], wfc_ref[0], (((1,), (0,)), ((), ())),
        precision=_PREC, preferred_element_type=jnp.float32)
    logits = logits + bfc_ref[0, 0, :][None, :]
    z = g * logits
    zm = z - jnp.max(z, axis=1, keepdims=True)
    ez = jnp.exp(zm)
    out_ref[0] = ez / jnp.sum(ez, axis=1, keepdims=True)


@functools.partial(jax.jit, static_argnames=())
def kernel(x, Wr, br, Ws, bs, Wdw, bdw, Wpw, bpw, Wfc, bfc):
    xf = x.reshape(B, K)
    wt = jnp.tile(Wr, (1, 1, 56, 56)).reshape(E, K)

    r, sel0, gate, idx, loss = pl.pallas_call(
        _router_kernel,
        grid=(KC,),
        in_specs=[
            pl.BlockSpec((B, KCHUNK), lambda k: (0, k)),
            pl.BlockSpec((E, KCHUNK), lambda k: (0, k)),
            pl.BlockSpec((1, E), lambda k: (0, 0)),
        ],
        out_specs=[
            pl.BlockSpec((B, E), lambda k: (0, 0)),
            pl.BlockSpec((B, E), lambda k: (0, 0)),
            pl.BlockSpec((B, 1), lambda k: (0, 0)),
            pl.BlockSpec((B, 1), lambda k: (0, 0)),
            pl.BlockSpec((1, 1), lambda k: (0, 0)),
        ],
        out_shape=[
            jax.ShapeDtypeStruct((B, E), jnp.float32),
            jax.ShapeDtypeStruct((B, E), jnp.float32),
            jax.ShapeDtypeStruct((B, 1), jnp.float32),
            jax.ShapeDtypeStruct((B, 1), jnp.int32),
            jax.ShapeDtypeStruct((1, 1), jnp.float32),
        ],
    )(xf, wt, br.reshape(1, E))
    del r

    x4 = x.reshape(B, 3, HO, 2, HO, 2).transpose(0, 1, 3, 5, 2, 4)
    x4 = x4.reshape(B, 12, HO, HO)
    x4p = jnp.pad(x4, ((0, 0), (0, 0), (0, 0), (0, WP - HO))
                  ).reshape(B, 12 * HO, WP)

    wsr = Ws.transpose(0, 1, 3, 4, 2).reshape(E, 32, 27)
    wdwr = Wdw.reshape(E, 32, 9).transpose(0, 2, 1)
    wpwr = Wpw.reshape(E, 64, 32)
    bs3 = bs.reshape(E, 1, 32)
    bdw3 = bdw.reshape(E, 1, 32)
    bpw3 = bpw.reshape(E, 1, 64)
    bfc3 = bfc.reshape(E, 1, NUM_CLASSES)

    idx_flat = idx.reshape(B)
    gate_flat = gate.reshape(B)

    grid_spec = pltpu.PrefetchScalarGridSpec(
        num_scalar_prefetch=2,
        grid=(B,),
        in_specs=[
            pl.BlockSpec((1, 12 * HO, WP), lambda b, i, g: (b, 0, 0)),
            pl.BlockSpec((1, 32, 27), lambda b, i, g: (i[b], 0, 0)),
            pl.BlockSpec((1, 1, 32), lambda b, i, g: (i[b], 0, 0)),
            pl.BlockSpec((1, 9, 32), lambda b, i, g: (i[b], 0, 0)),
            pl.BlockSpec((1, 1, 32), lambda b, i, g: (i[b], 0, 0)),
            pl.BlockSpec((1, 64, 32), lambda b, i, g: (i[b], 0, 0)),
            pl.BlockSpec((1, 1, 64), lambda b, i, g: (i[b], 0, 0)),
            pl.BlockSpec((1, 64, NUM_CLASSES), lambda b, i, g: (i[b], 0, 0)),
            pl.BlockSpec((1, 1, NUM_CLASSES), lambda b, i, g: (i[b], 0, 0)),
        ],
        out_specs=pl.BlockSpec((1, 1, NUM_CLASSES), lambda b, i, g: (b, 0, 0)),
    )
    out3 = pl.pallas_call(
        _expert_kernel,
        grid_spec=grid_spec,
        out_shape=jax.ShapeDtypeStruct((B, 1, NUM_CLASSES), jnp.float32),
    )(idx_flat, gate_flat, x4p, wsr, bs3, wdwr, bdw3, wpwr, bpw3, Wfc, bfc3)

    output = out3.reshape(B, NUM_CLASSES)
    return (output, sel0, loss.reshape(()), 0)


# speedup vs baseline: 6.1495x; 1.1044x over previous
import functools

import jax
import jax.numpy as jnp
from jax import lax
from jax.experimental import pallas as pl
from jax.experimental.pallas import tpu as pltpu

E = 8
B = 32
H = 224
HO = 112
WP = 128
NUM_CLASSES = 1000
K = 3 * H * H
KC = 8
KCHUNK = K // KC
_PREC = lax.Precision.HIGHEST
_EPREC = lax.Precision.DEFAULT


def _roll(v, shift, axis):
    return pltpu.roll(v, shift % v.shape[axis], axis)


def _router_kernel(x_ref, w_ref, br_ref, r_ref, sel0_ref, gate_ref,
                   idx_ref, loss_ref):
    k = pl.program_id(0)

    @pl.when(k == 0)
    def _init():
        r_ref[...] = jnp.zeros_like(r_ref)

    part = lax.dot_general(
        x_ref[...], w_ref[...], (((1,), (1,)), ((), ())),
        precision=_PREC, preferred_element_type=jnp.float32)
    r_ref[...] += part

    @pl.when(k == KC - 1)
    def _finish():
        r = r_ref[...] + 3136.0 * br_ref[...]
        m = jnp.max(r, axis=1, keepdims=True)
        ex = jnp.exp(r - m)
        s = jnp.sum(ex, axis=1, keepdims=True)
        sel = ex / s
        gate_ref[...] = jnp.max(sel, axis=1, keepdims=True)
        idx = jnp.argmax(r, axis=1)
        idx_ref[...] = idx[:, None].astype(jnp.int32)
        eiota = lax.broadcasted_iota(jnp.int32, (B, E), 1)
        sel0 = (eiota == idx[:, None]).astype(jnp.float32)
        sel0_ref[...] = sel0
        density = jnp.mean(sel0, axis=0)
        proxy = jnp.mean(sel, axis=0)
        loss = jnp.sum(proxy * density) * float(E)
        loss_ref[...] = jnp.reshape(loss, (1, 1))


def _expert_kernel(idx_ref, gate_ref, x_ref, ws_ref, bs_ref, wdw_ref,
                   bdw_ref, wpw_ref, bpw_ref, wfc_ref, bfc_ref, out_ref):
    del idx_ref
    b = pl.program_id(0)
    g = gate_ref[b]
    X = x_ref[0]

    riota = lax.broadcasted_iota(jnp.int32, (12 * HO, WP), 0)
    vmask = (riota % HO != HO - 1).astype(jnp.float32)
    Xv = _roll(X, -1, 0) * vmask
    Xh = _roll(X, -1, 1)
    Xvh = _roll(Xv, -1, 1)
    variants = {(0, 0): X, (1, 0): Xv, (0, 1): Xh, (1, 1): Xvh}

    taps = []
    for ki in range(3):
        for kj in range(3):
            V = variants[(ki // 2, kj // 2)]
            pi, pj = ki % 2, kj % 2
            for c in range(3):
                p = c * 4 + pi * 2 + pj
                taps.append(V[p * HO:(p + 1) * HO, :])
    P = jnp.stack(taps, axis=0).reshape(27, HO * WP)
    h1f = lax.dot_general(ws_ref[0], P, (((1,), (0,)), ((), ())),
                          precision=_EPREC, preferred_element_type=jnp.float32)
    fmask = (lax.broadcasted_iota(jnp.int32, (1, HO * WP), 1) % WP < HO
             ).astype(jnp.float32)
    colmask = (lax.broadcasted_iota(jnp.int32, (1, 1, WP), 2) < HO
               ).astype(jnp.float32)
    bs = bs_ref[0, 0, :]
    h1 = (jnp.maximum(h1f + bs[:, None], 0.0) * fmask).reshape(32, HO, WP)

    row3 = lax.broadcasted_iota(jnp.int32, (32, HO, WP), 1)
    top = (row3 != 0).astype(jnp.float32)
    bot = (row3 != HO - 1).astype(jnp.float32)
    vvar = {
        -1: _roll(h1, 1, 1) * top,
        0: h1,
        1: _roll(h1, -1, 1) * bot,
    }
    acc2 = jnp.zeros((32, HO, WP), jnp.float32)
    for di in (-1, 0, 1):
        for dj in (-1, 0, 1):
            Vt = vvar[di]
            if dj:
                Vt = _roll(Vt, -dj, 2)
            wv = wdw_ref[0, (di + 1) * 3 + (dj + 1), :]
            acc2 = acc2 + wv[:, None, None] * Vt
    bdw = bdw_ref[0, 0, :]
    h2 = jnp.maximum(acc2 + bdw[:, None, None], 0.0) * colmask

    h2f = h2.reshape(32, HO * WP)
    h3 = lax.dot_general(
        wpw_ref[0], h2f, (((1,), (0,)), ((), ())),
        precision=_EPREC, preferred_element_type=jnp.float32)
    bpw = bpw_ref[0, 0, :]
    h3 = jnp.maximum(h3 + bpw[:, None], 0.0)

    pvec = jnp.sum(h3 * fmask, axis=1) / float(HO * HO)

    logits = lax.dot_general(
        pvec[None, :], wfc_ref[0], (((1,), (0,)), ((), ())),
        precision=_EPREC, preferred_element_type=jnp.float32)
    logits = logits + bfc_ref[0, 0, :][None, :]
    z = g * logits
    zm = z - jnp.max(z, axis=1, keepdims=True)
    ez = jnp.exp(zm)
    out_ref[0] = ez / jnp.sum(ez, axis=1, keepdims=True)


@functools.partial(jax.jit, static_argnames=())
def kernel(x, Wr, br, Ws, bs, Wdw, bdw, Wpw, bpw, Wfc, bfc):
    xf = x.reshape(B, K)
    wt = jnp.tile(Wr, (1, 1, 56, 56)).reshape(E, K)

    r, sel0, gate, idx, loss = pl.pallas_call(
        _router_kernel,
        grid=(KC,),
        in_specs=[
            pl.BlockSpec((B, KCHUNK), lambda k: (0, k)),
            pl.BlockSpec((E, KCHUNK), lambda k: (0, k)),
            pl.BlockSpec((1, E), lambda k: (0, 0)),
        ],
        out_specs=[
            pl.BlockSpec((B, E), lambda k: (0, 0)),
            pl.BlockSpec((B, E), lambda k: (0, 0)),
            pl.BlockSpec((B, 1), lambda k: (0, 0)),
            pl.BlockSpec((B, 1), lambda k: (0, 0)),
            pl.BlockSpec((1, 1), lambda k: (0, 0)),
        ],
        out_shape=[
            jax.ShapeDtypeStruct((B, E), jnp.float32),
            jax.ShapeDtypeStruct((B, E), jnp.float32),
            jax.ShapeDtypeStruct((B, 1), jnp.float32),
            jax.ShapeDtypeStruct((B, 1), jnp.int32),
            jax.ShapeDtypeStruct((1, 1), jnp.float32),
        ],
    )(xf, wt, br.reshape(1, E))
    del r

    x4 = x.reshape(B, 3, HO, 2, HO, 2).transpose(0, 1, 3, 5, 2, 4)
    x4 = x4.reshape(B, 12, HO, HO)
    x4p = jnp.pad(x4, ((0, 0), (0, 0), (0, 0), (0, WP - HO))
                  ).reshape(B, 12 * HO, WP)

    wsr = Ws.transpose(0, 1, 3, 4, 2).reshape(E, 32, 27)
    wdwr = Wdw.reshape(E, 32, 9).transpose(0, 2, 1)
    wpwr = Wpw.reshape(E, 64, 32)
    bs3 = bs.reshape(E, 1, 32)
    bdw3 = bdw.reshape(E, 1, 32)
    bpw3 = bpw.reshape(E, 1, 64)
    bfc3 = bfc.reshape(E, 1, NUM_CLASSES)

    idx_flat = idx.reshape(B)
    gate_flat = gate.reshape(B)

    grid_spec = pltpu.PrefetchScalarGridSpec(
        num_scalar_prefetch=2,
        grid=(B,),
        in_specs=[
            pl.BlockSpec((1, 12 * HO, WP), lambda b, i, g: (b, 0, 0)),
            pl.BlockSpec((1, 32, 27), lambda b, i, g: (i[b], 0, 0)),
            pl.BlockSpec((1, 1, 32), lambda b, i, g: (i[b], 0, 0)),
            pl.BlockSpec((1, 9, 32), lambda b, i, g: (i[b], 0, 0)),
            pl.BlockSpec((1, 1, 32), lambda b, i, g: (i[b], 0, 0)),
            pl.BlockSpec((1, 64, 32), lambda b, i, g: (i[b], 0, 0)),
            pl.BlockSpec((1, 1, 64), lambda b, i, g: (i[b], 0, 0)),
            pl.BlockSpec((1, 64, NUM_CLASSES), lambda b, i, g: (i[b], 0, 0)),
            pl.BlockSpec((1, 1, NUM_CLASSES), lambda b, i, g: (i[b], 0, 0)),
        ],
        out_specs=pl.BlockSpec((1, 1, NUM_CLASSES), lambda b, i, g: (b, 0, 0)),
    )
    out3 = pl.pallas_call(
        _expert_kernel,
        grid_spec=grid_spec,
        out_shape=jax.ShapeDtypeStruct((B, 1, NUM_CLASSES), jnp.float32),
    )(idx_flat, gate_flat, x4p, wsr, bs3, wdwr, bdw3, wpwr, bpw3, Wfc, bfc3)

    output = out3.reshape(B, NUM_CLASSES)
    return (output, sel0, loss.reshape(()), 0)


# speedup vs baseline: 6.9435x; 1.1291x over previous
import functools

import jax
import jax.numpy as jnp
from jax import lax
from jax.experimental import pallas as pl
from jax.experimental.pallas import tpu as pltpu

E = 8
B = 32
H = 224
HO = 112
WP = 128
NUM_CLASSES = 1000
K = 3 * H * H
KC = 8
KCHUNK = K // KC
_PREC = lax.Precision.HIGHEST
_EPREC = lax.Precision.DEFAULT


def _roll(v, shift, axis):
    return pltpu.roll(v, shift % v.shape[axis], axis)


def _router_kernel(x_ref, w_ref, br_ref, r_ref, sel0_ref, gate_ref,
                   idx_ref, loss_ref):
    k = pl.program_id(0)

    @pl.when(k == 0)
    def _init():
        r_ref[...] = jnp.zeros_like(r_ref)

    part = lax.dot_general(
        x_ref[...], w_ref[...], (((1,), (1,)), ((), ())),
        precision=_PREC, preferred_element_type=jnp.float32)
    r_ref[...] += part

    @pl.when(k == KC - 1)
    def _finish():
        r = r_ref[...] + 3136.0 * br_ref[...]
        m = jnp.max(r, axis=1, keepdims=True)
        ex = jnp.exp(r - m)
        s = jnp.sum(ex, axis=1, keepdims=True)
        sel = ex / s
        gate_ref[...] = jnp.max(sel, axis=1, keepdims=True)
        idx = jnp.argmax(r, axis=1)
        idx_ref[...] = idx[:, None].astype(jnp.int32)
        eiota = lax.broadcasted_iota(jnp.int32, (B, E), 1)
        sel0 = (eiota == idx[:, None]).astype(jnp.float32)
        sel0_ref[...] = sel0
        density = jnp.mean(sel0, axis=0)
        proxy = jnp.mean(sel, axis=0)
        loss = jnp.sum(proxy * density) * float(E)
        loss_ref[...] = jnp.reshape(loss, (1, 1))


def _expert_kernel(idx_ref, gate_ref, x_ref, ws_ref, bs_ref, wdw_ref,
                   bdw_ref, wpw_ref, bpw_ref, wfc_ref, bfc_ref, out_ref):
    del idx_ref
    b = pl.program_id(0)
    g = gate_ref[b]
    X = x_ref[0]

    riota = lax.broadcasted_iota(jnp.int32, (12 * HO, WP), 0)
    vmask = (riota % HO != HO - 1).astype(jnp.float32)
    Xv = _roll(X, -1, 0) * vmask
    Xh = _roll(X, -1, 1)
    Xvh = _roll(Xv, -1, 1)
    variants = {(0, 0): X, (1, 0): Xv, (0, 1): Xh, (1, 1): Xvh}

    taps = []
    for ki in range(3):
        for kj in range(3):
            V = variants[(ki // 2, kj // 2)]
            pi, pj = ki % 2, kj % 2
            for c in range(3):
                p = c * 4 + pi * 2 + pj
                taps.append(V[p * HO:(p + 1) * HO, :])
    P = jnp.stack(taps, axis=0).reshape(27, HO * WP)
    h1f = lax.dot_general(ws_ref[0], P, (((1,), (0,)), ((), ())),
                          precision=_EPREC, preferred_element_type=jnp.float32)
    fmask = (lax.broadcasted_iota(jnp.int32, (1, HO * WP), 1) % WP < HO
             ).astype(jnp.float32)
    colmask = (lax.broadcasted_iota(jnp.int32, (1, 1, WP), 2) < HO
               ).astype(jnp.float32)
    bs = bs_ref[0, 0, :]
    h1 = (jnp.maximum(h1f + bs[:, None], 0.0) * fmask).reshape(32, HO, WP)

    row3 = lax.broadcasted_iota(jnp.int32, (32, HO, WP), 1)
    top = (row3 != 0).astype(jnp.float32)
    bot = (row3 != HO - 1).astype(jnp.float32)
    Hm = _roll(h1, 1, 2)
    Hp = _roll(h1, -1, 2)
    rrows = []
    for di in (-1, 0, 1):
        def w(dj):
            return wdw_ref[0, (di + 1) * 3 + (dj + 1), :][:, None, None]
        rrows.append(w(-1) * Hm + w(0) * h1 + w(1) * Hp)
    acc2 = (_roll(rrows[0], 1, 1) * top + rrows[1]
            + _roll(rrows[2], -1, 1) * bot)
    bdw = bdw_ref[0, 0, :]
    h2 = jnp.maximum(acc2 + bdw[:, None, None], 0.0) * colmask

    h2f = h2.reshape(32, HO * WP)
    h3 = lax.dot_general(
        wpw_ref[0], h2f, (((1,), (0,)), ((), ())),
        precision=_EPREC, preferred_element_type=jnp.float32)
    bpw = bpw_ref[0, 0, :]
    h3 = jnp.maximum(h3 + bpw[:, None], 0.0)

    pvec = jnp.sum(h3 * fmask, axis=1) / float(HO * HO)

    logits = lax.dot_general(
        pvec[None, :], wfc_ref[0], (((1,), (0,)), ((), ())),
        precision=_EPREC, preferred_element_type=jnp.float32)
    logits = logits + bfc_ref[0, 0, :][None, :]
    z = g * logits
    zm = z - jnp.max(z, axis=1, keepdims=True)
    ez = jnp.exp(zm)
    out_ref[0] = ez / jnp.sum(ez, axis=1, keepdims=True)


@functools.partial(jax.jit, static_argnames=())
def kernel(x, Wr, br, Ws, bs, Wdw, bdw, Wpw, bpw, Wfc, bfc):
    xf = x.reshape(B, K)
    wt = jnp.tile(Wr, (1, 1, 56, 56)).reshape(E, K)

    r, sel0, gate, idx, loss = pl.pallas_call(
        _router_kernel,
        grid=(KC,),
        in_specs=[
            pl.BlockSpec((B, KCHUNK), lambda k: (0, k)),
            pl.BlockSpec((E, KCHUNK), lambda k: (0, k)),
            pl.BlockSpec((1, E), lambda k: (0, 0)),
        ],
        out_specs=[
            pl.BlockSpec((B, E), lambda k: (0, 0)),
            pl.BlockSpec((B, E), lambda k: (0, 0)),
            pl.BlockSpec((B, 1), lambda k: (0, 0)),
            pl.BlockSpec((B, 1), lambda k: (0, 0)),
            pl.BlockSpec((1, 1), lambda k: (0, 0)),
        ],
        out_shape=[
            jax.ShapeDtypeStruct((B, E), jnp.float32),
            jax.ShapeDtypeStruct((B, E), jnp.float32),
            jax.ShapeDtypeStruct((B, 1), jnp.float32),
            jax.ShapeDtypeStruct((B, 1), jnp.int32),
            jax.ShapeDtypeStruct((1, 1), jnp.float32),
        ],
    )(xf, wt, br.reshape(1, E))
    del r

    x4 = x.reshape(B, 3, HO, 2, HO, 2).transpose(0, 1, 3, 5, 2, 4)
    x4 = x4.reshape(B, 12, HO, HO)
    x4p = jnp.pad(x4, ((0, 0), (0, 0), (0, 0), (0, WP - HO))
                  ).reshape(B, 12 * HO, WP)

    wsr = Ws.transpose(0, 1, 3, 4, 2).reshape(E, 32, 27)
    wdwr = Wdw.reshape(E, 32, 9).transpose(0, 2, 1)
    wpwr = Wpw.reshape(E, 64, 32)
    bs3 = bs.reshape(E, 1, 32)
    bdw3 = bdw.reshape(E, 1, 32)
    bpw3 = bpw.reshape(E, 1, 64)
    bfc3 = bfc.reshape(E, 1, NUM_CLASSES)

    idx_flat = idx.reshape(B)
    gate_flat = gate.reshape(B)

    grid_spec = pltpu.PrefetchScalarGridSpec(
        num_scalar_prefetch=2,
        grid=(B,),
        in_specs=[
            pl.BlockSpec((1, 12 * HO, WP), lambda b, i, g: (b, 0, 0)),
            pl.BlockSpec((1, 32, 27), lambda b, i, g: (i[b], 0, 0)),
            pl.BlockSpec((1, 1, 32), lambda b, i, g: (i[b], 0, 0)),
            pl.BlockSpec((1, 9, 32), lambda b, i, g: (i[b], 0, 0)),
            pl.BlockSpec((1, 1, 32), lambda b, i, g: (i[b], 0, 0)),
            pl.BlockSpec((1, 64, 32), lambda b, i, g: (i[b], 0, 0)),
            pl.BlockSpec((1, 1, 64), lambda b, i, g: (i[b], 0, 0)),
            pl.BlockSpec((1, 64, NUM_CLASSES), lambda b, i, g: (i[b], 0, 0)),
            pl.BlockSpec((1, 1, NUM_CLASSES), lambda b, i, g: (i[b], 0, 0)),
        ],
        out_specs=pl.BlockSpec((1, 1, NUM_CLASSES), lambda b, i, g: (b, 0, 0)),
    )
    out3 = pl.pallas_call(
        _expert_kernel,
        grid_spec=grid_spec,
        out_shape=jax.ShapeDtypeStruct((B, 1, NUM_CLASSES), jnp.float32),
    )(idx_flat, gate_flat, x4p, wsr, bs3, wdwr, bdw3, wpwr, bpw3, Wfc, bfc3)

    output = out3.reshape(B, NUM_CLASSES)
    return (output, sel0, loss.reshape(()), 0)


# speedup vs baseline: 7.0996x; 1.0225x over previous
import functools

import jax
import jax.numpy as jnp
from jax import lax
from jax.experimental import pallas as pl
from jax.experimental.pallas import tpu as pltpu

E = 8
B = 32
H = 224
HO = 112
WP = 128
NUM_CLASSES = 1000
K = 3 * H * H
KC = 8
KCHUNK = K // KC
SPS = 2
_PREC = lax.Precision.HIGHEST
_EPREC = lax.Precision.DEFAULT


def _roll(v, shift, axis):
    return pltpu.roll(v, shift % v.shape[axis], axis)


def _router_kernel(x_ref, w_ref, br_ref, r_ref, sel0_ref, gate_ref,
                   idx_ref, loss_ref):
    k = pl.program_id(0)

    @pl.when(k == 0)
    def _init():
        r_ref[...] = jnp.zeros_like(r_ref)

    part = lax.dot_general(
        x_ref[...], w_ref[...], (((1,), (1,)), ((), ())),
        precision=_PREC, preferred_element_type=jnp.float32)
    r_ref[...] += part

    @pl.when(k == KC - 1)
    def _finish():
        r = r_ref[...] + 3136.0 * br_ref[...]
        m = jnp.max(r, axis=1, keepdims=True)
        ex = jnp.exp(r - m)
        s = jnp.sum(ex, axis=1, keepdims=True)
        sel = ex / s
        gate_ref[...] = jnp.max(sel, axis=1, keepdims=True)
        idx = jnp.argmax(r, axis=1)
        idx_ref[...] = idx[:, None].astype(jnp.int32)
        eiota = lax.broadcasted_iota(jnp.int32, (B, E), 1)
        sel0 = (eiota == idx[:, None]).astype(jnp.float32)
        sel0_ref[...] = sel0
        density = jnp.mean(sel0, axis=0)
        proxy = jnp.mean(sel, axis=0)
        loss = jnp.sum(proxy * density) * float(E)
        loss_ref[...] = jnp.reshape(loss, (1, 1))


def _one_sample(X, g, ws, bs, wdw, bdw, wpw, bpw, wfc, bfc):
    riota = lax.broadcasted_iota(jnp.int32, (12 * HO, WP), 0)
    vmask = (riota % HO != HO - 1).astype(jnp.float32)
    Xv = _roll(X, -1, 0) * vmask
    Xh = _roll(X, -1, 1)
    Xvh = _roll(Xv, -1, 1)
    variants = {(0, 0): X, (1, 0): Xv, (0, 1): Xh, (1, 1): Xvh}

    taps = []
    for ki in range(3):
        for kj in range(3):
            V = variants[(ki // 2, kj // 2)]
            pi, pj = ki % 2, kj % 2
            for c in range(3):
                p = c * 4 + pi * 2 + pj
                taps.append(V[p * HO:(p + 1) * HO, :])
    P = jnp.stack(taps, axis=0).reshape(27, HO * WP)
    h1f = lax.dot_general(ws, P, (((1,), (0,)), ((), ())),
                          precision=_EPREC, preferred_element_type=jnp.float32)
    fmask = (lax.broadcasted_iota(jnp.int32, (1, HO * WP), 1) % WP < HO
             ).astype(jnp.float32)
    colmask = (lax.broadcasted_iota(jnp.int32, (1, 1, WP), 2) < HO
               ).astype(jnp.float32)
    h1 = (jnp.maximum(h1f + bs[:, None], 0.0) * fmask).reshape(32, HO, WP)

    row3 = lax.broadcasted_iota(jnp.int32, (32, HO, WP), 1)
    top = (row3 != 0).astype(jnp.float32)
    bot = (row3 != HO - 1).astype(jnp.float32)
    Hm = _roll(h1, 1, 2)
    Hp = _roll(h1, -1, 2)
    rrows = []
    for di in (-1, 0, 1):
        def w(dj):
            return wdw[(di + 1) * 3 + (dj + 1), :][:, None, None]
        rrows.append(w(-1) * Hm + w(0) * h1 + w(1) * Hp)
    acc2 = (_roll(rrows[0], 1, 1) * top + rrows[1]
            + _roll(rrows[2], -1, 1) * bot)
    h2 = jnp.maximum(acc2 + bdw[:, None, None], 0.0) * colmask

    h2f = h2.reshape(32, HO * WP)
    h3 = lax.dot_general(
        wpw, h2f, (((1,), (0,)), ((), ())),
        precision=_EPREC, preferred_element_type=jnp.float32)
    h3 = jnp.maximum(h3 + bpw[:, None], 0.0)

    pvec = jnp.sum(h3 * fmask, axis=1) / float(HO * HO)

    logits = lax.dot_general(
        pvec[None, :], wfc, (((1,), (0,)), ((), ())),
        precision=_EPREC, preferred_element_type=jnp.float32)
    logits = logits + bfc[None, :]
    z = g * logits
    zm = z - jnp.max(z, axis=1, keepdims=True)
    ez = jnp.exp(zm)
    return ez / jnp.sum(ez, axis=1, keepdims=True)


def _expert_kernel(idx_ref, gate_ref, x_ref,
                   ws0, ws1, bs0, bs1, wdw0, wdw1, bdw0, bdw1,
                   wpw0, wpw1, bpw0, bpw1, wfc0, wfc1, bfc0, bfc1, out_ref):
    del idx_ref
    b = pl.program_id(0)
    wrefs = ((ws0, bs0, wdw0, bdw0, wpw0, bpw0, wfc0, bfc0),
             (ws1, bs1, wdw1, bdw1, wpw1, bpw1, wfc1, bfc1))
    for s in range(SPS):
        ws, bs, wdw, bdw, wpw, bpw, wfc, bfc = wrefs[s]
        out_ref[s] = _one_sample(
            x_ref[s], gate_ref[b * SPS + s],
            ws[0], bs[0, 0, :], wdw[0], bdw[0, 0, :],
            wpw[0], bpw[0, 0, :], wfc[0], bfc[0, 0, :])


@functools.partial(jax.jit, static_argnames=())
def kernel(x, Wr, br, Ws, bs, Wdw, bdw, Wpw, bpw, Wfc, bfc):
    xf = x.reshape(B, K)
    wt = jnp.tile(Wr, (1, 1, 56, 56)).reshape(E, K)

    r, sel0, gate, idx, loss = pl.pallas_call(
        _router_kernel,
        grid=(KC,),
        in_specs=[
            pl.BlockSpec((B, KCHUNK), lambda k: (0, k)),
            pl.BlockSpec((E, KCHUNK), lambda k: (0, k)),
            pl.BlockSpec((1, E), lambda k: (0, 0)),
        ],
        out_specs=[
            pl.BlockSpec((B, E), lambda k: (0, 0)),
            pl.BlockSpec((B, E), lambda k: (0, 0)),
            pl.BlockSpec((B, 1), lambda k: (0, 0)),
            pl.BlockSpec((B, 1), lambda k: (0, 0)),
            pl.BlockSpec((1, 1), lambda k: (0, 0)),
        ],
        out_shape=[
            jax.ShapeDtypeStruct((B, E), jnp.float32),
            jax.ShapeDtypeStruct((B, E), jnp.float32),
            jax.ShapeDtypeStruct((B, 1), jnp.float32),
            jax.ShapeDtypeStruct((B, 1), jnp.int32),
            jax.ShapeDtypeStruct((1, 1), jnp.float32),
        ],
    )(xf, wt, br.reshape(1, E))
    del r

    x4 = x.reshape(B, 3, HO, 2, HO, 2).transpose(0, 1, 3, 5, 2, 4)
    x4 = x4.reshape(B, 12, HO, HO)
    x4p = jnp.pad(x4, ((0, 0), (0, 0), (0, 0), (0, WP - HO))
                  ).reshape(B, 12 * HO, WP)

    wsr = Ws.transpose(0, 1, 3, 4, 2).reshape(E, 32, 27)
    wdwr = Wdw.reshape(E, 32, 9).transpose(0, 2, 1)
    wpwr = Wpw.reshape(E, 64, 32)
    bs3 = bs.reshape(E, 1, 32)
    bdw3 = bdw.reshape(E, 1, 32)
    bpw3 = bpw.reshape(E, 1, 64)
    bfc3 = bfc.reshape(E, 1, NUM_CLASSES)

    idx_flat = idx.reshape(B)
    gate_flat = gate.reshape(B)

    def wspec(shape, s):
        return pl.BlockSpec(
            (1,) + shape,
            lambda b, i, g, s=s: (i[b * SPS + s],) + (0,) * len(shape))

    in_specs = [pl.BlockSpec((SPS, 12 * HO, WP), lambda b, i, g: (b, 0, 0))]
    for shape in [(32, 27), (1, 32), (9, 32), (1, 32), (64, 32), (1, 64),
                  (64, NUM_CLASSES), (1, NUM_CLASSES)]:
        for s in range(SPS):
            in_specs.append(wspec(shape, s))

    grid_spec = pltpu.PrefetchScalarGridSpec(
        num_scalar_prefetch=2,
        grid=(B // SPS,),
        in_specs=in_specs,
        out_specs=pl.BlockSpec((SPS, 1, NUM_CLASSES),
                               lambda b, i, g: (b, 0, 0)),
    )
    args = [idx_flat, gate_flat, x4p]
    for a in [wsr, bs3, wdwr, bdw3, wpwr, bpw3, Wfc, bfc3]:
        args.extend([a] * SPS)
    out3 = pl.pallas_call(
        _expert_kernel,
        grid_spec=grid_spec,
        out_shape=jax.ShapeDtypeStruct((B, 1, NUM_CLASSES), jnp.float32),
    )(*args)

    output = out3.reshape(B, NUM_CLASSES)
    return (output, sel0, loss.reshape(()), 0)


# speedup vs baseline: 7.7916x; 1.0975x over previous
import functools

import jax
import jax.numpy as jnp
from jax import lax
from jax.experimental import pallas as pl
from jax.experimental.pallas import tpu as pltpu

E = 8
B = 32
H = 224
HO = 112
WP = 128
NUM_CLASSES = 1000
K = 3 * H * H
KC = 8
KCHUNK = K // KC
SPS = 2
_PREC = lax.Precision.HIGHEST
_EPREC = lax.Precision.DEFAULT


def _roll(v, shift, axis):
    return pltpu.roll(v, shift % v.shape[axis], axis)


def _router_kernel(x_ref, w_ref, br_ref, r_ref, sel0_ref, gate_ref,
                   idx_ref, loss_ref):
    k = pl.program_id(0)

    @pl.when(k == 0)
    def _init():
        r_ref[...] = jnp.zeros_like(r_ref)

    part = lax.dot_general(
        x_ref[...], w_ref[...], (((1,), (1,)), ((), ())),
        precision=_PREC, preferred_element_type=jnp.float32)
    r_ref[...] += part

    @pl.when(k == KC - 1)
    def _finish():
        r = r_ref[...] + 3136.0 * br_ref[...]
        m = jnp.max(r, axis=1, keepdims=True)
        ex = jnp.exp(r - m)
        s = jnp.sum(ex, axis=1, keepdims=True)
        sel = ex / s
        gate_ref[...] = jnp.max(sel, axis=1, keepdims=True)
        idx = jnp.argmax(r, axis=1)
        idx_ref[...] = idx[:, None].astype(jnp.int32)
        eiota = lax.broadcasted_iota(jnp.int32, (B, E), 1)
        sel0 = (eiota == idx[:, None]).astype(jnp.float32)
        sel0_ref[...] = sel0
        density = jnp.mean(sel0, axis=0)
        proxy = jnp.mean(sel, axis=0)
        loss = jnp.sum(proxy * density) * float(E)
        loss_ref[...] = jnp.reshape(loss, (1, 1))


def _one_sample(X, g, ws, bs, wdw, bdw, wpw, bpw, wfc, bfc):
    riota = lax.broadcasted_iota(jnp.int32, (12 * HO, WP), 0)
    vmask = (riota % HO != HO - 1).astype(jnp.float32)
    Xv = _roll(X, -1, 0) * vmask
    Xh = _roll(X, -1, 1)
    Xvh = _roll(Xv, -1, 1)
    variants = {(0, 0): X, (1, 0): Xv, (0, 1): Xh, (1, 1): Xvh}

    taps = []
    for ki in range(3):
        for kj in range(3):
            V = variants[(ki // 2, kj // 2)]
            pi, pj = ki % 2, kj % 2
            for c in range(3):
                p = c * 4 + pi * 2 + pj
                taps.append(V[p * HO:(p + 1) * HO, :])
    bf = jnp.bfloat16
    P = jnp.stack(taps, axis=0).reshape(27, HO * WP)
    h1f = lax.dot_general(ws, P, (((1,), (0,)), ((), ())),
                          precision=_EPREC,
                          preferred_element_type=jnp.float32)
    fmask = (lax.broadcasted_iota(jnp.int32, (1, HO * WP), 1) % WP < HO
             ).astype(bf)
    colmask = (lax.broadcasted_iota(jnp.int32, (1, 1, WP), 2) < HO
               ).astype(bf)
    h1 = (jnp.maximum(h1f + bs[:, None], 0.0).astype(bf) * fmask
          ).reshape(32, HO, WP)

    row3 = lax.broadcasted_iota(jnp.int32, (32, HO, WP), 1)
    top = (row3 != 0).astype(bf)
    bot = (row3 != HO - 1).astype(bf)
    Hm = _roll(h1, 1, 2)
    Hp = _roll(h1, -1, 2)
    rrows = []
    for di in (-1, 0, 1):
        def w(dj):
            return wdw[(di + 1) * 3 + (dj + 1), :][:, None, None].astype(bf)
        rrows.append(w(-1) * Hm + w(0) * h1 + w(1) * Hp)
    acc2 = (_roll(rrows[0], 1, 1) * top + rrows[1]
            + _roll(rrows[2], -1, 1) * bot)
    h2 = jnp.maximum(acc2 + bdw[:, None, None].astype(bf), bf(0)) * colmask

    h2f = h2.reshape(32, HO * WP)
    h3 = lax.dot_general(
        wpw.astype(bf), h2f, (((1,), (0,)), ((), ())),
        precision=_EPREC, preferred_element_type=jnp.float32)
    h3 = jnp.maximum(h3 + bpw[:, None], 0.0)

    npad = float(HO * (WP - HO))
    pvec = ((jnp.sum(h3, axis=1) - npad * jnp.maximum(bpw, 0.0))
            / float(HO * HO))

    logits = lax.dot_general(
        pvec[None, :], wfc, (((1,), (0,)), ((), ())),
        precision=_EPREC, preferred_element_type=jnp.float32)
    logits = logits + bfc[None, :]
    z = g * logits
    zm = z - jnp.max(z, axis=1, keepdims=True)
    ez = jnp.exp(zm)
    return ez / jnp.sum(ez, axis=1, keepdims=True)


def _expert_kernel(idx_ref, gate_ref, x_ref,
                   ws0, ws1, bs0, bs1, wdw0, wdw1, bdw0, bdw1,
                   wpw0, wpw1, bpw0, bpw1, wfc0, wfc1, bfc0, bfc1, out_ref):
    del idx_ref
    b = pl.program_id(0)
    wrefs = ((ws0, bs0, wdw0, bdw0, wpw0, bpw0, wfc0, bfc0),
             (ws1, bs1, wdw1, bdw1, wpw1, bpw1, wfc1, bfc1))
    for s in range(SPS):
        ws, bs, wdw, bdw, wpw, bpw, wfc, bfc = wrefs[s]
        out_ref[s] = _one_sample(
            x_ref[s], gate_ref[b * SPS + s],
            ws[0], bs[0, 0, :], wdw[0], bdw[0, 0, :],
            wpw[0], bpw[0, 0, :], wfc[0], bfc[0, 0, :])


@functools.partial(jax.jit, static_argnames=())
def kernel(x, Wr, br, Ws, bs, Wdw, bdw, Wpw, bpw, Wfc, bfc):
    xf = x.reshape(B, K)
    wt = jnp.tile(Wr, (1, 1, 56, 56)).reshape(E, K)

    r, sel0, gate, idx, loss = pl.pallas_call(
        _router_kernel,
        grid=(KC,),
        in_specs=[
            pl.BlockSpec((B, KCHUNK), lambda k: (0, k)),
            pl.BlockSpec((E, KCHUNK), lambda k: (0, k)),
            pl.BlockSpec((1, E), lambda k: (0, 0)),
        ],
        out_specs=[
            pl.BlockSpec((B, E), lambda k: (0, 0)),
            pl.BlockSpec((B, E), lambda k: (0, 0)),
            pl.BlockSpec((B, 1), lambda k: (0, 0)),
            pl.BlockSpec((B, 1), lambda k: (0, 0)),
            pl.BlockSpec((1, 1), lambda k: (0, 0)),
        ],
        out_shape=[
            jax.ShapeDtypeStruct((B, E), jnp.float32),
            jax.ShapeDtypeStruct((B, E), jnp.float32),
            jax.ShapeDtypeStruct((B, 1), jnp.float32),
            jax.ShapeDtypeStruct((B, 1), jnp.int32),
            jax.ShapeDtypeStruct((1, 1), jnp.float32),
        ],
    )(xf, wt, br.reshape(1, E))
    del r

    x4 = x.reshape(B, 3, HO, 2, HO, 2).transpose(0, 1, 3, 5, 2, 4)
    x4 = x4.reshape(B, 12, HO, HO)
    x4p = jnp.pad(x4, ((0, 0), (0, 0), (0, 0), (0, WP - HO))
                  ).reshape(B, 12 * HO, WP)

    wsr = Ws.transpose(0, 1, 3, 4, 2).reshape(E, 32, 27)
    wdwr = Wdw.reshape(E, 32, 9).transpose(0, 2, 1)
    wpwr = Wpw.reshape(E, 64, 32)
    bs3 = bs.reshape(E, 1, 32)
    bdw3 = bdw.reshape(E, 1, 32)
    bpw3 = bpw.reshape(E, 1, 64)
    bfc3 = bfc.reshape(E, 1, NUM_CLASSES)

    idx_flat = idx.reshape(B)
    gate_flat = gate.reshape(B)

    def wspec(shape, s):
        return pl.BlockSpec(
            (1,) + shape,
            lambda b, i, g, s=s: (i[b * SPS + s],) + (0,) * len(shape))

    in_specs = [pl.BlockSpec((SPS, 12 * HO, WP), lambda b, i, g: (b, 0, 0))]
    for shape in [(32, 27), (1, 32), (9, 32), (1, 32), (64, 32), (1, 64),
                  (64, NUM_CLASSES), (1, NUM_CLASSES)]:
        for s in range(SPS):
            in_specs.append(wspec(shape, s))

    grid_spec = pltpu.PrefetchScalarGridSpec(
        num_scalar_prefetch=2,
        grid=(B // SPS,),
        in_specs=in_specs,
        out_specs=pl.BlockSpec((SPS, 1, NUM_CLASSES),
                               lambda b, i, g: (b, 0, 0)),
    )
    args = [idx_flat, gate_flat, x4p]
    for a in [wsr, bs3, wdwr, bdw3, wpwr, bpw3, Wfc, bfc3]:
        args.extend([a] * SPS)
    out3 = pl.pallas_call(
        _expert_kernel,
        grid_spec=grid_spec,
        out_shape=jax.ShapeDtypeStruct((B, 1, NUM_CLASSES), jnp.float32),
    )(*args)

    output = out3.reshape(B, NUM_CLASSES)
    return (output, sel0, loss.reshape(()), 0)
